# baseline (speedup 1.0000x reference)
import jax
import jax.numpy as jnp
from jax.experimental import pallas as pl
from jax.experimental.pallas import tpu as pltpu

EMBED_DIM = 4096
NUM_EXPERTS = 64
NUM_CLASSES = 2
B = 4
SEQ = 2048
EXPERTS_PER_CLASS = NUM_EXPERTS // NUM_CLASSES
M_BLK = 512
SEQ_BLKS = SEQ // M_BLK


def _router_kernel(cls_ref, x_ref, w_ref, b_ref, out_ref, wt_bf):
    m = pl.program_id(0)

    @pl.when(m == 0)
    def _():
        wt_bf[...] = w_ref[...].astype(jnp.bfloat16)

    batch = (m * M_BLK) // SEQ
    cls = cls_ref[batch]
    xb = x_ref[...].astype(jnp.bfloat16)
    logits = jax.lax.dot_general(
        wt_bf[...], xb, (((1,), (1,)), ((), ())),
        preferred_element_type=jnp.float32)
    logits = logits + b_ref[...].T
    e = jax.lax.broadcasted_iota(jnp.int32, logits.shape, 0)
    in_class = (e // EXPERTS_PER_CLASS) == cls
    logits = jnp.where(in_class, logits, -jnp.inf)
    mx = jnp.max(logits, axis=0, keepdims=True)
    ex = jnp.exp(logits - mx)
    weights = ex / jnp.sum(ex, axis=0, keepdims=True)
    out_ref[...] = weights[None]


def kernel(x, class_label, W, b):
    x2d = x.reshape(B * SEQ, EMBED_DIM)
    b2d = b.reshape(1, NUM_EXPERTS)
    cls_i32 = class_label.astype(jnp.int32)
    grid = (B * SEQ) // M_BLK
    out = pl.pallas_call(
        _router_kernel,
        grid_spec=pltpu.PrefetchScalarGridSpec(
            num_scalar_prefetch=1,
            grid=(grid,),
            in_specs=[
                pl.BlockSpec((M_BLK, EMBED_DIM), lambda m, c: (m, 0)),
                pl.BlockSpec((NUM_EXPERTS, EMBED_DIM), lambda m, c: (0, 0)),
                pl.BlockSpec((1, NUM_EXPERTS), lambda m, c: (0, 0)),
            ],
            out_specs=pl.BlockSpec(
                (1, NUM_EXPERTS, M_BLK),
                lambda m, c: (m // SEQ_BLKS, 0, m % SEQ_BLKS),
            ),
            scratch_shapes=[
                pltpu.VMEM((NUM_EXPERTS, EMBED_DIM), jnp.bfloat16),
            ],
        ),
        out_shape=jax.ShapeDtypeStruct((B, NUM_EXPERTS, SEQ), jnp.float32),
        compiler_params=pltpu.CompilerParams(
            dimension_semantics=("parallel",),
        ),
    )(cls_i32, x2d, W, b2d)
    return jnp.swapaxes(out, 1, 2)
